# baseline (speedup 1.0000x reference)
import functools

import jax
import jax.numpy as jnp
from jax import lax
from jax.experimental import pallas as pl
from jax.experimental.pallas import tpu as pltpu
from jax.experimental.pallas import tpu_sc as plsc

VOCAB = 1000000
D_MODEL = 64
BATCH = 4096
HIST = 200

NC, NS = 2, 16
NW = NC * NS
BB = BATCH // NW
NV = BB // 16
NCH = HIST
NBUF = 5
NGROUP = NCH // NBUF

_mesh = plsc.VectorSubcoreMesh(core_axis_name="c", subcore_axis_name="s")


@functools.partial(
    pl.kernel,
    out_type=jax.ShapeDtypeStruct((HIST, D_MODEL, BATCH), jnp.float32),
    mesh=_mesh,
    scratch_types=[
        pltpu.VMEM((NCH, BB), jnp.int32),
        pltpu.VMEM((NBUF, BB, D_MODEL), jnp.float32),
        pltpu.VMEM((NBUF, D_MODEL, BB), jnp.float32),
    ] + [pltpu.SemaphoreType.DMA] * (2 * NBUF),
    compiler_params=pltpu.CompilerParams(use_tc_tiling_on_sc=False,
                                         needs_layout_passes=False),
)
def _embed_sc(x_hbm, tab_hbm, out_hbm, idx_v, rows_v, trows_v, *sems):
    gsems = sems[:NBUF]
    wsems = sems[NBUF:]
    wid = lax.axis_index("s") * NC + lax.axis_index("c")
    pltpu.sync_copy(x_hbm.at[:, pl.ds(wid * BB, BB)], idx_v)

    def fire_gather(j, b):
        for k in range(NV):
            idx16 = idx_v[j, pl.ds(k * 16, 16)]
            pltpu.async_copy(tab_hbm.at[idx16],
                             rows_v.at[b, pl.ds(k * 16, 16)], gsems[b])

    def wait_gather(b):
        pltpu.make_async_copy(tab_hbm.at[idx_v.at[0]], rows_v.at[b],
                              gsems[b]).wait()

    def wait_writeback(b):
        pltpu.make_async_copy(trows_v.at[b],
                              out_hbm.at[0, :, pl.ds(0, BB)], wsems[b]).wait()

    row_ids = [lax.broadcasted_iota(jnp.int32, (16,), 0) + g * 16
               for g in range(NV)]

    cols = [jnp.full((16,), jf, jnp.int32) for jf in range(D_MODEL)]

    def transpose_chunk(b):
        for jf in range(D_MODEL):
            for g in range(NV):
                vals = plsc.load_gather(rows_v.at[b], [row_ids[g], cols[jf]])
                trows_v[b, jf, pl.ds(g * 16, 16)] = vals

    for b in range(NBUF):
        fire_gather(b, b)

    def group(g, carry):
        for b in range(NBUF):
            j = g * NBUF + b
            wait_gather(b)

            @pl.when(j >= NBUF)
            def _():
                wait_writeback(b)

            transpose_chunk(b)

            @pl.when(j + NBUF < NCH)
            def _():
                fire_gather(j + NBUF, b)

            pltpu.async_copy(trows_v.at[b],
                             out_hbm.at[j, :, pl.ds(wid * BB, BB)], wsems[b])

        return carry

    lax.fori_loop(0, NGROUP, group, 0)

    for b in range(NBUF):
        wait_writeback(b)


def kernel(x, embed_weight):
    xt = jnp.swapaxes(x, 0, 1).astype(jnp.int32)
    out_t = _embed_sc(xt, embed_weight)
    return jnp.transpose(out_t, (2, 0, 1))


# speedup vs baseline: 1.8727x; 1.8727x over previous
import functools

import jax
import jax.numpy as jnp
from jax import lax
from jax.experimental import pallas as pl
from jax.experimental.pallas import tpu as pltpu
from jax.experimental.pallas import tpu_sc as plsc

VOCAB = 1000000
D_MODEL = 64
BATCH = 4096
HIST = 200

NC, NS = 2, 16
NW = NC * NS
BB = BATCH // NW
NV = BB // 16
NCH = HIST
NBUF = 5
NGROUP = NCH // NBUF

_mesh = plsc.VectorSubcoreMesh(core_axis_name="c", subcore_axis_name="s")


@functools.partial(
    pl.kernel,
    out_type=jax.ShapeDtypeStruct((HIST, D_MODEL, BATCH), jnp.float32),
    mesh=_mesh,
    scratch_types=[
        pltpu.VMEM((NCH, BB), jnp.int32),
        pltpu.VMEM((NBUF, BB, D_MODEL), jnp.float32),
        pltpu.VMEM((NBUF, D_MODEL, BB), jnp.float32),
    ] + [pltpu.SemaphoreType.DMA] * (2 * NBUF),
    compiler_params=pltpu.CompilerParams(use_tc_tiling_on_sc=False,
                                         needs_layout_passes=False),
)
def _embed_sc(x_hbm, tab_hbm, out_hbm, idx_v, rows_v, trows_v, *sems):
    gsems = sems[:NBUF]
    wsems = sems[NBUF:]
    wid = lax.axis_index("s") * NC + lax.axis_index("c")
    pltpu.sync_copy(x_hbm.at[:, pl.ds(wid * BB, BB)], idx_v)

    def fire_gather(j, b):
        for k in range(NV):
            idx16 = idx_v[j, pl.ds(k * 16, 16)]
            pltpu.async_copy(tab_hbm.at[idx16],
                             rows_v.at[b, pl.ds(k * 16, 16)], gsems[b])

    def wait_gather(b):
        pltpu.make_async_copy(tab_hbm.at[idx_v.at[0]], rows_v.at[b],
                              gsems[b]).wait()

    def wait_writeback(b):
        pltpu.make_async_copy(trows_v.at[b],
                              out_hbm.at[0, :, pl.ds(0, BB)], wsems[b]).wait()

    row_ids = [lax.broadcasted_iota(jnp.int32, (16,), 0) + g * 16
               for g in range(NV)]

    iota16 = lax.broadcasted_iota(jnp.int32, (16,), 0)

    def transpose_chunk(b):
        def diag(d, carry):
            for jb in range(D_MODEL // 16):
                colv = ((d + iota16) & 15) + jb * 16
                for g in range(NV):
                    rowv = row_ids[g]
                    vals = plsc.load_gather(rows_v.at[b], [rowv, colv])
                    plsc.store_scatter(trows_v.at[b], [colv, rowv], vals)
            return carry

        lax.fori_loop(0, 16, diag, 0)

    for b in range(NBUF):
        fire_gather(b, b)

    def group(g, carry):
        for b in range(NBUF):
            j = g * NBUF + b
            wait_gather(b)

            @pl.when(j >= NBUF)
            def _():
                wait_writeback(b)

            transpose_chunk(b)

            @pl.when(j + NBUF < NCH)
            def _():
                fire_gather(j + NBUF, b)

            pltpu.async_copy(trows_v.at[b],
                             out_hbm.at[j, :, pl.ds(wid * BB, BB)], wsems[b])

        return carry

    lax.fori_loop(0, NGROUP, group, 0)

    for b in range(NBUF):
        wait_writeback(b)


def kernel(x, embed_weight):
    xt = jnp.swapaxes(x, 0, 1).astype(jnp.int32)
    out_t = _embed_sc(xt, embed_weight)
    return jnp.transpose(out_t, (2, 0, 1))


# speedup vs baseline: 1.8816x; 1.0047x over previous
import functools

import jax
import jax.numpy as jnp
from jax import lax
from jax.experimental import pallas as pl
from jax.experimental.pallas import tpu as pltpu
from jax.experimental.pallas import tpu_sc as plsc

VOCAB = 1000000
D_MODEL = 64
BATCH = 4096
HIST = 200

N = BATCH * HIST
NC, NS = 2, 16
NW = NC * NS
PER_W = N // NW
CH = 128
NV = CH // 16
NCH = PER_W // CH
NBUF = 10
LAG = 4
NGROUP = NCH // NBUF

_mesh = plsc.VectorSubcoreMesh(core_axis_name="c", subcore_axis_name="s")


@functools.partial(
    pl.kernel,
    out_type=jax.ShapeDtypeStruct((NW * NCH, CH, D_MODEL), jnp.float32),
    mesh=_mesh,
    scratch_types=[
        pltpu.VMEM((NCH, CH), jnp.int32),
        pltpu.VMEM((NBUF, CH, D_MODEL), jnp.float32),
    ] + [pltpu.SemaphoreType.DMA] * (2 * NBUF),
    compiler_params=pltpu.CompilerParams(use_tc_tiling_on_sc=False),
)
def _embed_sc(x_hbm, tab_hbm, out_hbm, idx_v, rows_v, *sems):
    gsems = sems[:NBUF]
    wsems = sems[NBUF:]
    wid = lax.axis_index("s") * NC + lax.axis_index("c")
    pltpu.sync_copy(x_hbm.at[wid], idx_v)

    def fire_gather(j, b):
        for k in range(NV):
            idx16 = idx_v[j, pl.ds(k * 16, 16)]
            pltpu.async_copy(tab_hbm.at[idx16],
                             rows_v.at[b, pl.ds(k * 16, 16)], gsems[b])

    def wait_gather(b):
        pltpu.make_async_copy(tab_hbm.at[idx_v.at[0]], rows_v.at[b],
                              gsems[b]).wait()

    def wait_writeback(b):
        pltpu.make_async_copy(rows_v.at[b], out_hbm.at[0], wsems[b]).wait()

    for b in range(NBUF):
        fire_gather(b, b)

    def group(g, carry):
        for b in range(NBUF):
            j = g * NBUF + b
            wait_gather(b)
            pltpu.async_copy(rows_v.at[b], out_hbm.at[wid * NCH + j], wsems[b])

            bw = (b - LAG) % NBUF

            @pl.when(jnp.logical_and(j - LAG >= 0, j - LAG + NBUF < NCH))
            def _():
                wait_writeback(bw)
                fire_gather(j - LAG + NBUF, bw)

        return carry

    lax.fori_loop(0, NGROUP, group, 0)

    for b in range(NBUF):
        wait_writeback(b)


def kernel(x, embed_weight):
    xf = x.reshape(-1).astype(jnp.int32).reshape(NW, NCH, CH)
    out = _embed_sc(xf, embed_weight)
    return out.reshape(BATCH, HIST, D_MODEL)
